# Initial kernel scaffold; baseline (speedup 1.0000x reference)
import jax
import jax.numpy as jnp
from jax.experimental import pallas as pl


def kernel(feature, edge_index, W, b):
    raise NotImplementedError("write your pallas kernel here")


# speedup vs baseline: 5.4271x; 5.4271x over previous
import functools

import jax
import jax.numpy as jnp
from jax import lax
from jax.experimental import pallas as pl
from jax.experimental.pallas import tpu as pltpu
from jax.experimental.pallas import tpu_sc as plsc

N_NODES = 10000
D_FEAT = 128
N_EDGES = 320000

NC = 2
NS = 16
E_PER_TILE = N_EDGES // (NC * NS)
CHUNK = 80
N_CHUNKS = E_PER_TILE // CHUNK
N_PAD = 10240
ROWS_PER_TILE = N_PAD // NS
ZROWS = 128


def _sc_segment_sum(feature, src, dst):
    mesh = plsc.VectorSubcoreMesh(core_axis_name="c", subcore_axis_name="s")

    @functools.partial(
        pl.kernel,
        out_type=jax.ShapeDtypeStruct((NC, N_PAD, D_FEAT), jnp.float32),
        mesh=mesh,
        scratch_types=[
            pltpu.VMEM((CHUNK,), jnp.int32),
            pltpu.VMEM((CHUNK,), jnp.int32),
            pltpu.VMEM((CHUNK, D_FEAT), jnp.float32),
            pltpu.VMEM((ZROWS, D_FEAT), jnp.float32),
            pltpu.VMEM_SHARED((N_PAD, D_FEAT), jnp.float32),
            pltpu.SemaphoreType.DMA,
        ],
    )
    def k(feature_hbm, src_hbm, dst_hbm, out_hbm, src_v, dst_v, rows_v,
          zbuf_v, acc_sh, sem):
        c = lax.axis_index("c")
        s = lax.axis_index("s")

        zv = jnp.zeros((16,), jnp.float32)

        def zfill(i, _):
            r = i // (D_FEAT // 16)
            col = (i % (D_FEAT // 16)) * 16
            zbuf_v[r, pl.ds(col, 16)] = zv
            return 0

        lax.fori_loop(0, ZROWS * (D_FEAT // 16), zfill, 0)

        row0 = s * ROWS_PER_TILE
        for rep in range(ROWS_PER_TILE // ZROWS):
            pltpu.sync_copy(zbuf_v, acc_sh.at[pl.ds(row0 + rep * ZROWS, ZROWS)])

        plsc.subcore_barrier()

        base = (c * NS + s) * E_PER_TILE

        def body(j, _):
            off = pl.multiple_of(base + j * CHUNK, 8)
            pltpu.sync_copy(src_hbm.at[pl.ds(off, CHUNK)], src_v)
            pltpu.sync_copy(dst_hbm.at[pl.ds(off, CHUNK)], dst_v)
            pltpu.async_copy(feature_hbm.at[src_v], rows_v, sem).wait()
            pltpu.sync_copy(rows_v, acc_sh.at[dst_v], add=True)
            return 0

        lax.fori_loop(0, N_CHUNKS, body, 0)

        plsc.subcore_barrier()

        pltpu.sync_copy(
            acc_sh.at[pl.ds(row0, ROWS_PER_TILE)],
            out_hbm.at[c, pl.ds(row0, ROWS_PER_TILE)],
        )

    return k(feature, src, dst)


def _tc_apply(p0, p1, feature, W, b2d):
    BR = 2000

    def body(p0_ref, p1_ref, f_ref, w_ref, b_ref, o_ref):
        agg = p0_ref[...] + p1_ref[...]
        z = jnp.dot(agg, w_ref[...], preferred_element_type=jnp.float32)
        o_ref[...] = jnp.maximum(z + b_ref[...], 0.0) + f_ref[...]

    return pl.pallas_call(
        body,
        grid=(N_NODES // BR,),
        in_specs=[
            pl.BlockSpec((BR, D_FEAT), lambda i: (i, 0)),
            pl.BlockSpec((BR, D_FEAT), lambda i: (i, 0)),
            pl.BlockSpec((BR, D_FEAT), lambda i: (i, 0)),
            pl.BlockSpec((D_FEAT, D_FEAT), lambda i: (0, 0)),
            pl.BlockSpec((1, D_FEAT), lambda i: (0, 0)),
        ],
        out_specs=pl.BlockSpec((BR, D_FEAT), lambda i: (i, 0)),
        out_shape=jax.ShapeDtypeStruct((N_NODES, D_FEAT), jnp.float32),
    )(p0, p1, feature, W, b2d)


def kernel(feature, edge_index, W, b):
    src = edge_index[0]
    dst = edge_index[1]
    partials = _sc_segment_sum(feature, src, dst)
    return _tc_apply(partials[0, :N_NODES], partials[1, :N_NODES], feature, W,
                     b.reshape(1, D_FEAT))
